# baseline (speedup 1.0000x reference)
import jax
import jax.numpy as jnp
from jax import lax
from jax.experimental import pallas as pl
from jax.experimental.pallas import tpu as pltpu
from jax.experimental.pallas import tpu_sc as plsc

P = 196
SPLIT = 128
NSC = P - SPLIT
NW = 32
NBUF = 6


def _sc_body(in_hbm, *rest):
    outs = rest[:NSC]
    buf = rest[NSC]
    cid = lax.axis_index("c")
    sid = lax.axis_index("s")
    wid = sid * 2 + cid
    pltpu.sync_copy(in_hbm.at[:, :, pl.ds(0, SPLIT)], buf)
    for k in range(NSC):
        @pl.when(wid == (k % NW))
        def _(k=k):
            pltpu.sync_copy(buf, outs[k].at[:, :, pl.ds(0, SPLIT)])


_sc_call = pl.kernel(
    _sc_body,
    out_type=tuple(jax.ShapeDtypeStruct((4, 192, P - 1), jnp.float32)
                   for _ in range(NSC)),
    mesh=plsc.VectorSubcoreMesh(core_axis_name="c", subcore_axis_name="s"),
    scratch_types=[pltpu.VMEM((4, 192, SPLIT), jnp.float32)],
    compiler_params=pltpu.CompilerParams(use_tc_tiling_on_sc=True),
)


def _tc_main_body(in_ref, *rest):
    out_refs = rest[:SPLIT]
    scratch = rest[SPLIT:SPLIT + NBUF]
    sems = rest[SPLIT + NBUF:SPLIT + 2 * NBUF]
    a = in_ref[:, :, 0:P - 1]
    b = in_ref[:, :, 1:P]
    lane = lax.broadcasted_iota(jnp.int32, (4, 192, P - 1), 2)
    dmas = [None] * NBUF
    for i in range(SPLIT):
        p = i % NBUF
        if dmas[p] is not None:
            dmas[p].wait()
        scratch[p][...] = jnp.where(lane < i, a, b)
        dma = pltpu.make_async_copy(scratch[p], out_refs[i], sems[p])
        dma.start()
        dmas[p] = dma
    for p in range(NBUF):
        if dmas[p] is not None:
            dmas[p].wait()


_tc_main = pl.pallas_call(
    _tc_main_body,
    in_specs=[pl.BlockSpec(memory_space=pltpu.VMEM)],
    out_specs=tuple(pl.BlockSpec(memory_space=pl.ANY) for _ in range(SPLIT)),
    out_shape=tuple(jax.ShapeDtypeStruct((4, 192, P - 1), jnp.float32)
                    for _ in range(SPLIT)),
    scratch_shapes=([pltpu.VMEM((4, 192, P - 1), jnp.float32)] * NBUF
                    + [pltpu.SemaphoreType.DMA] * NBUF),
)


RW = P - 1 - SPLIT


def _tc_fin_body(in_ref, *rest):
    out_refs = rest[NSC:2 * NSC]
    scratch = rest[2 * NSC:2 * NSC + NBUF]
    sems = rest[2 * NSC + NBUF:2 * NSC + 2 * NBUF]
    a = in_ref[:, :, SPLIT:P - 1]
    b = in_ref[:, :, SPLIT + 1:P]
    lane = lax.broadcasted_iota(jnp.int32, (4, 192, RW), 2) + SPLIT
    dmas = [None] * NBUF
    for k in range(NSC):
        i = SPLIT + k
        p = k % NBUF
        if dmas[p] is not None:
            dmas[p].wait()
        scratch[p][...] = jnp.where(lane < i, a, b)
        dma = pltpu.make_async_copy(
            scratch[p], out_refs[k].at[:, :, pl.ds(SPLIT, RW)], sems[p])
        dma.start()
        dmas[p] = dma
    for p in range(NBUF):
        if dmas[p] is not None:
            dmas[p].wait()


_tc_fin = pl.pallas_call(
    _tc_fin_body,
    in_specs=([pl.BlockSpec(memory_space=pltpu.VMEM)]
              + [pl.BlockSpec(memory_space=pl.ANY) for _ in range(NSC)]),
    out_specs=tuple(pl.BlockSpec(memory_space=pl.ANY) for _ in range(NSC)),
    out_shape=tuple(jax.ShapeDtypeStruct((4, 192, P - 1), jnp.float32)
                    for _ in range(NSC)),
    input_output_aliases={k + 1: k for k in range(NSC)},
    scratch_shapes=([pltpu.VMEM((4, 192, RW), jnp.float32)] * NBUF
                    + [pltpu.SemaphoreType.DMA] * NBUF),
)


def kernel(patches):
    pt = jnp.transpose(patches, (1, 2, 0))
    partials = _sc_call(pt)
    low = _tc_main(pt)
    high = _tc_fin(pt, *partials)
    outs = tuple(low) + tuple(high)
    return tuple(jnp.transpose(o, (2, 0, 1)) for o in outs)


# speedup vs baseline: 1.7287x; 1.7287x over previous
import jax
import jax.numpy as jnp
from jax import lax
from jax.experimental import pallas as pl
from jax.experimental.pallas import tpu as pltpu

P = 196


NBUF = 6


def _body(in_ref, *rest):
    out_refs = rest[:P]
    scratch = rest[P:P + NBUF]
    sems = rest[P + NBUF:P + 2 * NBUF]
    a = in_ref[:, :, 0:P - 1]
    b = in_ref[:, :, 1:P]
    lane = lax.broadcasted_iota(jnp.int32, (4, 192, P - 1), 2)
    dmas = [None] * NBUF
    for i in range(P):
        p = i % NBUF
        if dmas[p] is not None:
            dmas[p].wait()
        scratch[p][...] = jnp.where(lane < i, a, b)
        dma = pltpu.make_async_copy(scratch[p], out_refs[i], sems[p])
        dma.start()
        dmas[p] = dma
    for p in range(NBUF):
        if dmas[p] is not None:
            dmas[p].wait()


_call = pl.pallas_call(
    _body,
    in_specs=[pl.BlockSpec(memory_space=pltpu.VMEM)],
    out_specs=tuple(pl.BlockSpec(memory_space=pl.ANY) for _ in range(P)),
    out_shape=tuple(jax.ShapeDtypeStruct((4, 192, P - 1), jnp.float32)
                    for _ in range(P)),
    scratch_shapes=([pltpu.VMEM((4, 192, P - 1), jnp.float32)] * NBUF
                    + [pltpu.SemaphoreType.DMA] * NBUF),
)


def kernel(patches):
    pt = jnp.transpose(patches, (1, 2, 0))
    outs = _call(pt)
    return tuple(jnp.transpose(o, (2, 0, 1)) for o in outs)
